# baseline (speedup 1.0000x reference)
import functools

import jax
import jax.numpy as jnp
from jax import lax
from jax.experimental import pallas as pl
from jax.experimental.pallas import tpu as pltpu
from jax.experimental.pallas import tpu_sc as plsc

N_NODES = 10000
D_FEAT = 256
N_EDGES = 160000
HIDDEN = 16
N_CLASSES = 7

NC, NS, LANES = 2, 16, 16
NW = NC * NS
NP = 10240
TRASH = N_NODES
BB = 128
EDGES_PER_W = 5120
NB = EDGES_PER_W // BB
NBUF = 8
ROWS_PER_S = NP // NS

def _wid():
    return lax.axis_index("s") * NC + lax.axis_index("c")


PER_W = N_EDGES // NW
FULL_ROWS = PER_W // BB
TAIL = PER_W - FULL_ROWS * BB


@functools.cache
def _sc_degree_fn():
    mesh = plsc.VectorSubcoreMesh(core_axis_name="c", subcore_axis_name="s",
                                  num_cores=NC, num_subcores=NS)

    @functools.partial(
        pl.kernel,
        out_type=[
            jax.ShapeDtypeStruct((NC, NP, LANES), jnp.float32),
            jax.ShapeDtypeStruct((NW, NB, BB), jnp.int32),
            jax.ShapeDtypeStruct((NW, NB, BB), jnp.int32),
        ],
        mesh=mesh,
        compiler_params=pltpu.CompilerParams(use_tc_tiling_on_sc=False),
        scratch_types=[
            pltpu.VMEM((NB, BB), jnp.int32),
            pltpu.VMEM((NB, BB), jnp.int32),
            pltpu.VMEM((BB, LANES), jnp.float32),
            pltpu.VMEM((ROWS_PER_S, LANES), jnp.float32),
            pltpu.VMEM_SHARED((NP, LANES), jnp.float32),
            pltpu.SemaphoreType.DMA,
            pltpu.SemaphoreType.DMA,
        ],
    )
    def _sc_degree(zeros_hbm, ones_hbm, ei_hbm, out_hbm, src_o, dst_o,
                   srcv, dstv, onesv, bounce, acc, lsem, ssem):
        c = lax.axis_index("c")
        s = lax.axis_index("s")
        wid = _wid()
        base = s * ROWS_PER_S
        ebase = wid * PER_W
        for j in range(FULL_ROWS):
            pltpu.async_copy(
                ei_hbm.at[0, pl.ds(ebase + j * BB, BB)], srcv.at[j], lsem)
            pltpu.async_copy(
                ei_hbm.at[1, pl.ds(ebase + j * BB, BB)], dstv.at[j], ssem)
        pltpu.async_copy(
            ei_hbm.at[0, pl.ds(ebase + FULL_ROWS * BB, TAIL)],
            srcv.at[FULL_ROWS, pl.ds(0, TAIL)], lsem)
        pltpu.async_copy(
            ei_hbm.at[1, pl.ds(ebase + FULL_ROWS * BB, TAIL)],
            dstv.at[FULL_ROWS, pl.ds(0, TAIL)], ssem)
        pltpu.sync_copy(zeros_hbm.at[pl.ds(base, ROWS_PER_S)], bounce)
        pltpu.sync_copy(bounce, acc.at[pl.ds(base, ROWS_PER_S)])
        pltpu.sync_copy(ones_hbm, onesv)
        for j in range(FULL_ROWS):
            pltpu.make_async_copy(
                ei_hbm.at[0, pl.ds(ebase + j * BB, BB)], srcv.at[j],
                lsem).wait()
            pltpu.make_async_copy(
                ei_hbm.at[1, pl.ds(ebase + j * BB, BB)], dstv.at[j],
                ssem).wait()
        pltpu.make_async_copy(
            ei_hbm.at[0, pl.ds(ebase + FULL_ROWS * BB, TAIL)],
            srcv.at[FULL_ROWS, pl.ds(0, TAIL)], lsem).wait()
        pltpu.make_async_copy(
            ei_hbm.at[1, pl.ds(ebase + FULL_ROWS * BB, TAIL)],
            dstv.at[FULL_ROWS, pl.ds(0, TAIL)], ssem).wait()
        iota = lax.iota(jnp.int32, LANES)
        for k in range(1, BB // LANES):
            trash = TRASH + iota + (LANES * (k - 1)) % (NP - N_NODES - LANES)
            srcv[FULL_ROWS, pl.ds(k * LANES, LANES)] = trash
            dstv[FULL_ROWS, pl.ds(k * LANES, LANES)] = trash
        head_s = srcv[FULL_ROWS, pl.ds(0, LANES)]
        head_d = dstv[FULL_ROWS, pl.ds(0, LANES)]
        tail_trash = TRASH + iota + 112
        srcv[FULL_ROWS, pl.ds(0, LANES)] = jnp.where(
            iota < TAIL, head_s, tail_trash)
        dstv[FULL_ROWS, pl.ds(0, LANES)] = jnp.where(
            iota < TAIL, head_d, tail_trash)
        pltpu.async_copy(srcv, src_o.at[wid], lsem)
        pltpu.async_copy(dstv, dst_o.at[wid], lsem)
        plsc.subcore_barrier()

        for jo in range(NB // NBUF):
            for b in range(NBUF):
                j = jo * NBUF + b
                pltpu.async_copy(onesv, acc.at[srcv.at[j]], ssem, add=True)
            for b in range(NBUF):
                j = jo * NBUF + b
                pltpu.make_async_copy(onesv, acc.at[srcv.at[j]], ssem).wait()
        pltpu.make_async_copy(srcv, src_o.at[wid], lsem).wait()
        pltpu.make_async_copy(dstv, dst_o.at[wid], lsem).wait()
        plsc.subcore_barrier()
        pltpu.sync_copy(acc.at[pl.ds(base, ROWS_PER_S)], bounce)
        pltpu.sync_copy(bounce, out_hbm.at[c, pl.ds(base, ROWS_PER_S)])

    return _sc_degree


@functools.cache
def _sc_aggregate_fn():
    mesh = plsc.VectorSubcoreMesh(core_axis_name="c", subcore_axis_name="s",
                                  num_cores=NC, num_subcores=NS)

    @functools.partial(
        pl.kernel,
        out_type=jax.ShapeDtypeStruct((NC, NP, LANES), jnp.float32),
        mesh=mesh,
        compiler_params=pltpu.CompilerParams(use_tc_tiling_on_sc=False),
        scratch_types=[
            pltpu.VMEM((NB, BB), jnp.int32),
            pltpu.VMEM((NB, BB), jnp.int32),
            pltpu.VMEM((NBUF, BB, LANES), jnp.float32),
            pltpu.VMEM((ROWS_PER_S, LANES), jnp.float32),
            pltpu.VMEM_SHARED((NP, LANES), jnp.float32),
            pltpu.SemaphoreType.DMA((NBUF,)),
            pltpu.SemaphoreType.DMA((NBUF,)),
        ],
    )
    def _sc_aggregate(g_hbm, src_hbm, dst_hbm, out_hbm, srcv, dstv, bufs,
                      bounce, acc, gsem, ssem):
        c = lax.axis_index("c")
        s = lax.axis_index("s")
        base = s * ROWS_PER_S
        pltpu.sync_copy(g_hbm.at[pl.ds(base, ROWS_PER_S)], bounce)
        pltpu.sync_copy(bounce, acc.at[pl.ds(base, ROWS_PER_S)])
        pltpu.sync_copy(src_hbm.at[_wid()], srcv)
        pltpu.sync_copy(dst_hbm.at[_wid()], dstv)
        plsc.subcore_barrier()

        for b in range(NBUF):
            pltpu.async_copy(g_hbm.at[srcv.at[b]], bufs.at[b], gsem.at[b])

        for jo in range(NB // NBUF - 1):
            for b in range(NBUF):
                j = jo * NBUF + b
                pltpu.make_async_copy(
                    g_hbm.at[srcv.at[j]], bufs.at[b], gsem.at[b]).wait()
                pltpu.async_copy(
                    bufs.at[b], acc.at[dstv.at[j]], ssem.at[b], add=True)
            for b in range(NBUF):
                j = jo * NBUF + b
                pltpu.make_async_copy(
                    bufs.at[b], acc.at[dstv.at[j]], ssem.at[b]).wait()
                pltpu.async_copy(
                    g_hbm.at[srcv.at[j + NBUF]], bufs.at[b], gsem.at[b])
        for b in range(NBUF):
            j = NB - NBUF + b
            pltpu.make_async_copy(
                g_hbm.at[srcv.at[j]], bufs.at[b], gsem.at[b]).wait()
            pltpu.async_copy(
                bufs.at[b], acc.at[dstv.at[j]], ssem.at[b], add=True)
        for b in range(NBUF):
            j = NB - NBUF + b
            pltpu.make_async_copy(
                bufs.at[b], acc.at[dstv.at[j]], ssem.at[b]).wait()
        plsc.subcore_barrier()
        pltpu.sync_copy(acc.at[pl.ds(base, ROWS_PER_S)], bounce)
        pltpu.sync_copy(bounce, out_hbm.at[c, pl.ds(base, ROWS_PER_S)])

    return _sc_aggregate


NPK = NP // 8


def _tc_matmul1_body(x_ref, w1_ref, b1_ref, lin_ref):
    lin = lax.dot_general(x_ref[...], w1_ref[...], (((1,), (1,)), ((), ())),
                          preferred_element_type=jnp.float32) + b1_ref[...]
    lin_ref[pl.ds(0, N_NODES), :] = lin
    lin_ref[pl.ds(N_NODES, NP - N_NODES), :] = jnp.zeros(
        (NP - N_NODES, LANES), jnp.float32)


def _tc_matmul1(x, W1, b1):
    return pl.pallas_call(
        _tc_matmul1_body,
        out_shape=jax.ShapeDtypeStruct((NP, LANES), jnp.float32),
    )(x, W1, b1)


def _disp(degp_ref):
    return lax.rsqrt(1.0 + degp_ref[0] + degp_ref[1])


def _tc_scale1_body(lin_ref, degp_ref, g1_ref):
    g1_ref[...] = lin_ref[...] * _disp(degp_ref)


def _tc_scale1(linp, degpp):
    return pl.pallas_call(
        _tc_scale1_body,
        out_shape=jax.ShapeDtypeStruct((NPK, 128), jnp.float32),
    )(linp, degpp)


def _tc_fuse2_body(g1_ref, p_ref, degp_ref, bw2_ref, b2_ref, g2_ref):
    dis = _disp(degp_ref)
    h1 = jnp.maximum(dis * (p_ref[0] + p_ref[1] - g1_ref[...]), 0.0)
    lin = lax.dot_general(h1, bw2_ref[...], (((1,), (0,)), ((), ())),
                          preferred_element_type=jnp.float32) + b2_ref[...]
    g2_ref[...] = lin * dis


def _tc_fuse2(g1p, pp, degpp, BW2, b2t):
    return pl.pallas_call(
        _tc_fuse2_body,
        out_shape=jax.ShapeDtypeStruct((NPK, 128), jnp.float32),
    )(g1p, pp, degpp, BW2, b2t)


def _tc_fuse3_body(g2_ref, q_ref, degp_ref, bones_ref, mask_ref, out_ref):
    mask = mask_ref[...]
    bones = bones_ref[...]
    z = _disp(degp_ref) * (q_ref[0] + q_ref[1] - g2_ref[...])
    gsum = lax.dot_general(z * mask, bones, (((1,), (0,)), ((), ())),
                           preferred_element_type=jnp.float32)
    m = gsum * (1.0 / N_CLASSES)
    e = jnp.exp(jnp.clip(z - m, -80.0, 80.0)) * mask
    se = lax.dot_general(e, bones, (((1,), (0,)), ((), ())),
                         preferred_element_type=jnp.float32)
    out_ref[...] = z - m - jnp.log(se)


def _tc_fuse3(g2p, qp, degpp, BO, maskp):
    return pl.pallas_call(
        _tc_fuse3_body,
        out_shape=jax.ShapeDtypeStruct((NPK, 128), jnp.float32),
    )(g2p, qp, degpp, BO, maskp)


def kernel(x, edge_index, W1, b1, W2, b2):
    ei32 = edge_index.astype(jnp.int32)
    zeros = jnp.zeros((NP, LANES), jnp.float32)
    ones = jnp.ones((BB, LANES), jnp.float32)
    W2p = jnp.zeros((LANES, HIDDEN), jnp.float32).at[:N_CLASSES].set(W2)
    BW2 = jnp.kron(jnp.eye(8, dtype=jnp.float32), W2p.T)
    b2t = jnp.tile(jnp.zeros((LANES,), jnp.float32).at[:N_CLASSES].set(b2),
                   8).reshape(1, 128)
    BO = jnp.kron(jnp.eye(8, dtype=jnp.float32),
                  jnp.ones((LANES, LANES), jnp.float32))
    maskp = jnp.tile(
        (jnp.arange(LANES) < N_CLASSES).astype(jnp.float32), 8).reshape(1, 128)
    b1r = b1.reshape(1, HIDDEN)

    def pack(a):
        return a.reshape(a.shape[:-2] + (NPK, 128))

    def unpack(a):
        return a.reshape(a.shape[:-2] + (NP, LANES))

    sc_degree = _sc_degree_fn()
    sc_aggregate = _sc_aggregate_fn()
    degp, src_p, dst_p = sc_degree(zeros, ones, ei32)
    degpp = pack(degp)
    linp = pack(_tc_matmul1(x, W1, b1r))
    g1p = _tc_scale1(linp, degpp)
    pp = pack(sc_aggregate(unpack(g1p), src_p, dst_p))
    g2p = _tc_fuse2(g1p, pp, degpp, BW2, b2t)
    qp = pack(sc_aggregate(unpack(g2p), src_p, dst_p))
    out = unpack(_tc_fuse3(g2p, qp, degpp, BO, maskp))
    return out[:N_NODES, :N_CLASSES]


# speedup vs baseline: 1.0039x; 1.0039x over previous
import functools

import jax
import jax.numpy as jnp
from jax import lax
from jax.experimental import pallas as pl
from jax.experimental.pallas import tpu as pltpu
from jax.experimental.pallas import tpu_sc as plsc

N_NODES = 10000
D_FEAT = 256
N_EDGES = 160000
HIDDEN = 16
N_CLASSES = 7

NC, NS, LANES = 2, 16, 16
NW = NC * NS
NP = 10240
TRASH = N_NODES
BB = 128
EDGES_PER_W = 5120
NB = EDGES_PER_W // BB
NBUF = 8
ROWS_PER_S = NP // NS

def _wid():
    return lax.axis_index("s") * NC + lax.axis_index("c")


PER_W = N_EDGES // NW
FULL_ROWS = PER_W // BB
TAIL = PER_W - FULL_ROWS * BB


@functools.cache
def _sc_degree_fn():
    mesh = plsc.VectorSubcoreMesh(core_axis_name="c", subcore_axis_name="s",
                                  num_cores=NC, num_subcores=NS)

    @functools.partial(
        pl.kernel,
        out_type=[
            jax.ShapeDtypeStruct((NC, NP, LANES), jnp.float32),
            jax.ShapeDtypeStruct((NW, NB, BB), jnp.int32),
            jax.ShapeDtypeStruct((NW, NB, BB), jnp.int32),
        ],
        mesh=mesh,
        compiler_params=pltpu.CompilerParams(use_tc_tiling_on_sc=False),
        scratch_types=[
            pltpu.VMEM((NB, BB), jnp.int32),
            pltpu.VMEM((NB, BB), jnp.int32),
            pltpu.VMEM((BB, LANES), jnp.float32),
            pltpu.VMEM((ROWS_PER_S, LANES), jnp.float32),
            pltpu.VMEM_SHARED((NP, LANES), jnp.float32),
            pltpu.SemaphoreType.DMA,
            pltpu.SemaphoreType.DMA,
        ],
    )
    def _sc_degree(zeros_hbm, ones_hbm, ei_hbm, out_hbm, src_o, dst_o,
                   srcv, dstv, onesv, bounce, acc, lsem, ssem):
        c = lax.axis_index("c")
        s = lax.axis_index("s")
        wid = _wid()
        base = s * ROWS_PER_S
        ebase = wid * PER_W
        for j in range(FULL_ROWS):
            pltpu.async_copy(
                ei_hbm.at[0, pl.ds(ebase + j * BB, BB)], srcv.at[j], lsem)
            pltpu.async_copy(
                ei_hbm.at[1, pl.ds(ebase + j * BB, BB)], dstv.at[j], ssem)
        pltpu.async_copy(
            ei_hbm.at[0, pl.ds(ebase + FULL_ROWS * BB, TAIL)],
            srcv.at[FULL_ROWS, pl.ds(0, TAIL)], lsem)
        pltpu.async_copy(
            ei_hbm.at[1, pl.ds(ebase + FULL_ROWS * BB, TAIL)],
            dstv.at[FULL_ROWS, pl.ds(0, TAIL)], ssem)
        pltpu.sync_copy(zeros_hbm, bounce)
        pltpu.sync_copy(bounce, acc.at[pl.ds(base, ROWS_PER_S)])
        pltpu.sync_copy(ones_hbm, onesv)
        for j in range(FULL_ROWS):
            pltpu.make_async_copy(
                ei_hbm.at[0, pl.ds(ebase + j * BB, BB)], srcv.at[j],
                lsem).wait()
            pltpu.make_async_copy(
                ei_hbm.at[1, pl.ds(ebase + j * BB, BB)], dstv.at[j],
                ssem).wait()
        pltpu.make_async_copy(
            ei_hbm.at[0, pl.ds(ebase + FULL_ROWS * BB, TAIL)],
            srcv.at[FULL_ROWS, pl.ds(0, TAIL)], lsem).wait()
        pltpu.make_async_copy(
            ei_hbm.at[1, pl.ds(ebase + FULL_ROWS * BB, TAIL)],
            dstv.at[FULL_ROWS, pl.ds(0, TAIL)], ssem).wait()
        iota = lax.iota(jnp.int32, LANES)
        for k in range(1, BB // LANES):
            trash = TRASH + iota + (LANES * (k - 1)) % (NP - N_NODES - LANES)
            srcv[FULL_ROWS, pl.ds(k * LANES, LANES)] = trash
            dstv[FULL_ROWS, pl.ds(k * LANES, LANES)] = trash
        head_s = srcv[FULL_ROWS, pl.ds(0, LANES)]
        head_d = dstv[FULL_ROWS, pl.ds(0, LANES)]
        tail_trash = TRASH + iota + 112
        srcv[FULL_ROWS, pl.ds(0, LANES)] = jnp.where(
            iota < TAIL, head_s, tail_trash)
        dstv[FULL_ROWS, pl.ds(0, LANES)] = jnp.where(
            iota < TAIL, head_d, tail_trash)
        pltpu.async_copy(srcv, src_o.at[wid], lsem)
        pltpu.async_copy(dstv, dst_o.at[wid], lsem)
        plsc.subcore_barrier()

        for jo in range(NB // NBUF):
            for b in range(NBUF):
                j = jo * NBUF + b
                pltpu.async_copy(onesv, acc.at[srcv.at[j]], ssem, add=True)
            for b in range(NBUF):
                j = jo * NBUF + b
                pltpu.make_async_copy(onesv, acc.at[srcv.at[j]], ssem).wait()
        pltpu.make_async_copy(srcv, src_o.at[wid], lsem).wait()
        pltpu.make_async_copy(dstv, dst_o.at[wid], lsem).wait()
        plsc.subcore_barrier()
        pltpu.sync_copy(acc.at[pl.ds(base, ROWS_PER_S)], bounce)
        pltpu.sync_copy(bounce, out_hbm.at[c, pl.ds(base, ROWS_PER_S)])

    return _sc_degree


@functools.cache
def _sc_aggregate_fn():
    mesh = plsc.VectorSubcoreMesh(core_axis_name="c", subcore_axis_name="s",
                                  num_cores=NC, num_subcores=NS)

    @functools.partial(
        pl.kernel,
        out_type=jax.ShapeDtypeStruct((NC, NP, LANES), jnp.float32),
        mesh=mesh,
        compiler_params=pltpu.CompilerParams(use_tc_tiling_on_sc=False),
        scratch_types=[
            pltpu.VMEM((NB, BB), jnp.int32),
            pltpu.VMEM((NB, BB), jnp.int32),
            pltpu.VMEM((NBUF, BB, LANES), jnp.float32),
            pltpu.VMEM((ROWS_PER_S, LANES), jnp.float32),
            pltpu.VMEM_SHARED((NP, LANES), jnp.float32),
            pltpu.SemaphoreType.DMA((NBUF,)),
            pltpu.SemaphoreType.DMA((NBUF,)),
        ],
    )
    def _sc_aggregate(g_hbm, src_hbm, dst_hbm, out_hbm, srcv, dstv, bufs,
                      bounce, acc, gsem, ssem):
        c = lax.axis_index("c")
        s = lax.axis_index("s")
        base = s * ROWS_PER_S
        pltpu.sync_copy(g_hbm.at[pl.ds(base, ROWS_PER_S)], bounce)
        pltpu.sync_copy(bounce, acc.at[pl.ds(base, ROWS_PER_S)])
        pltpu.sync_copy(src_hbm.at[_wid()], srcv)
        pltpu.sync_copy(dst_hbm.at[_wid()], dstv)
        plsc.subcore_barrier()

        for b in range(NBUF):
            pltpu.async_copy(g_hbm.at[srcv.at[b]], bufs.at[b], gsem.at[b])

        for jo in range(NB // NBUF - 1):
            for b in range(NBUF):
                j = jo * NBUF + b
                pltpu.make_async_copy(
                    g_hbm.at[srcv.at[j]], bufs.at[b], gsem.at[b]).wait()
                pltpu.async_copy(
                    bufs.at[b], acc.at[dstv.at[j]], ssem.at[b], add=True)
            for b in range(NBUF):
                j = jo * NBUF + b
                pltpu.make_async_copy(
                    bufs.at[b], acc.at[dstv.at[j]], ssem.at[b]).wait()
                pltpu.async_copy(
                    g_hbm.at[srcv.at[j + NBUF]], bufs.at[b], gsem.at[b])
        for b in range(NBUF):
            j = NB - NBUF + b
            pltpu.make_async_copy(
                g_hbm.at[srcv.at[j]], bufs.at[b], gsem.at[b]).wait()
            pltpu.async_copy(
                bufs.at[b], acc.at[dstv.at[j]], ssem.at[b], add=True)
        for b in range(NBUF):
            j = NB - NBUF + b
            pltpu.make_async_copy(
                bufs.at[b], acc.at[dstv.at[j]], ssem.at[b]).wait()
        plsc.subcore_barrier()
        pltpu.sync_copy(acc.at[pl.ds(base, ROWS_PER_S)], bounce)
        pltpu.sync_copy(bounce, out_hbm.at[c, pl.ds(base, ROWS_PER_S)])

    return _sc_aggregate


NPK = NP // 8


def _tc_matmul1_body(x_ref, w1_ref, b1_ref, lin_ref):
    lin = lax.dot_general(x_ref[...], w1_ref[...], (((1,), (1,)), ((), ())),
                          preferred_element_type=jnp.float32) + b1_ref[...]
    lin_ref[pl.ds(0, N_NODES), :] = lin
    lin_ref[pl.ds(N_NODES, NP - N_NODES), :] = jnp.zeros(
        (NP - N_NODES, LANES), jnp.float32)


def _tc_matmul1(x, W1, b1):
    return pl.pallas_call(
        _tc_matmul1_body,
        out_shape=jax.ShapeDtypeStruct((NP, LANES), jnp.float32),
    )(x, W1, b1)


def _disp(degp_ref):
    return lax.rsqrt(1.0 + degp_ref[0] + degp_ref[1])


def _tc_scale1_body(lin_ref, degp_ref, g1_ref):
    g1_ref[...] = lin_ref[...] * _disp(degp_ref)


def _tc_scale1(linp, degpp):
    return pl.pallas_call(
        _tc_scale1_body,
        out_shape=jax.ShapeDtypeStruct((NPK, 128), jnp.float32),
    )(linp, degpp)


def _tc_fuse2_body(g1_ref, p_ref, degp_ref, bw2_ref, b2_ref, g2_ref):
    dis = _disp(degp_ref)
    h1 = jnp.maximum(dis * (p_ref[0] + p_ref[1] - g1_ref[...]), 0.0)
    lin = lax.dot_general(h1, bw2_ref[...], (((1,), (0,)), ((), ())),
                          preferred_element_type=jnp.float32) + b2_ref[...]
    g2_ref[...] = lin * dis


def _tc_fuse2(g1p, pp, degpp, BW2, b2t):
    return pl.pallas_call(
        _tc_fuse2_body,
        out_shape=jax.ShapeDtypeStruct((NPK, 128), jnp.float32),
    )(g1p, pp, degpp, BW2, b2t)


def _tc_fuse3_body(g2_ref, q_ref, degp_ref, bones_ref, mask_ref, out_ref):
    mask = mask_ref[...]
    bones = bones_ref[...]
    z = _disp(degp_ref) * (q_ref[0] + q_ref[1] - g2_ref[...])
    gsum = lax.dot_general(z * mask, bones, (((1,), (0,)), ((), ())),
                           preferred_element_type=jnp.float32)
    m = gsum * (1.0 / N_CLASSES)
    e = jnp.exp(jnp.clip(z - m, -80.0, 80.0)) * mask
    se = lax.dot_general(e, bones, (((1,), (0,)), ((), ())),
                         preferred_element_type=jnp.float32)
    out_ref[...] = z - m - jnp.log(se)


def _tc_fuse3(g2p, qp, degpp, BO, maskp):
    return pl.pallas_call(
        _tc_fuse3_body,
        out_shape=jax.ShapeDtypeStruct((NPK, 128), jnp.float32),
    )(g2p, qp, degpp, BO, maskp)


def kernel(x, edge_index, W1, b1, W2, b2):
    ei32 = edge_index.astype(jnp.int32)
    zeros = jnp.zeros((ROWS_PER_S, LANES), jnp.float32)
    ones = jnp.ones((BB, LANES), jnp.float32)
    W2p = jnp.zeros((LANES, HIDDEN), jnp.float32).at[:N_CLASSES].set(W2)
    BW2 = jnp.kron(jnp.eye(8, dtype=jnp.float32), W2p.T)
    b2t = jnp.tile(jnp.zeros((LANES,), jnp.float32).at[:N_CLASSES].set(b2),
                   8).reshape(1, 128)
    BO = jnp.kron(jnp.eye(8, dtype=jnp.float32),
                  jnp.ones((LANES, LANES), jnp.float32))
    maskp = jnp.tile(
        (jnp.arange(LANES) < N_CLASSES).astype(jnp.float32), 8).reshape(1, 128)
    b1r = b1.reshape(1, HIDDEN)

    def pack(a):
        return a.reshape(a.shape[:-2] + (NPK, 128))

    def unpack(a):
        return a.reshape(a.shape[:-2] + (NP, LANES))

    sc_degree = _sc_degree_fn()
    sc_aggregate = _sc_aggregate_fn()
    degp, src_p, dst_p = sc_degree(zeros, ones, ei32)
    degpp = pack(degp)
    linp = pack(_tc_matmul1(x, W1, b1r))
    g1p = _tc_scale1(linp, degpp)
    pp = pack(sc_aggregate(unpack(g1p), src_p, dst_p))
    g2p = _tc_fuse2(g1p, pp, degpp, BW2, b2t)
    qp = pack(sc_aggregate(unpack(g2p), src_p, dst_p))
    outp = _tc_fuse3(g2p, qp, degpp, BO, maskp)
    return outp[:N_NODES // 8].reshape(N_NODES, LANES)[:, :N_CLASSES]


# speedup vs baseline: 1.0267x; 1.0228x over previous
import functools

import jax
import jax.numpy as jnp
from jax import lax
from jax.experimental import pallas as pl
from jax.experimental.pallas import tpu as pltpu
from jax.experimental.pallas import tpu_sc as plsc

N_NODES = 10000
D_FEAT = 256
N_EDGES = 160000
HIDDEN = 16
N_CLASSES = 7

NC, NS, LANES = 2, 16, 16
NW = NC * NS
NP = 10240
TRASH = N_NODES
BB = 128
EDGES_PER_W = 5120
NB = EDGES_PER_W // BB
NBUF = 8
ROWS_PER_S = NP // NS

def _wid():
    return lax.axis_index("s") * NC + lax.axis_index("c")


PER_W = N_EDGES // NW
FULL_ROWS = PER_W // BB
TAIL = PER_W - FULL_ROWS * BB


@functools.cache
def _sc_degree_fn():
    mesh = plsc.VectorSubcoreMesh(core_axis_name="c", subcore_axis_name="s",
                                  num_cores=NC, num_subcores=NS)

    @functools.partial(
        pl.kernel,
        out_type=[
            jax.ShapeDtypeStruct((NC, NP, LANES), jnp.float32),
            jax.ShapeDtypeStruct((NW, NB, BB), jnp.int32),
            jax.ShapeDtypeStruct((NW, NB, BB), jnp.int32),
        ],
        mesh=mesh,
        compiler_params=pltpu.CompilerParams(use_tc_tiling_on_sc=False),
        scratch_types=[
            pltpu.VMEM((NB, BB), jnp.int32),
            pltpu.VMEM((NB, BB), jnp.int32),
            pltpu.VMEM((BB, LANES), jnp.float32),
            pltpu.VMEM((ROWS_PER_S, LANES), jnp.float32),
            pltpu.VMEM_SHARED((NP, LANES), jnp.float32),
            pltpu.SemaphoreType.DMA,
            pltpu.SemaphoreType.DMA,
        ],
    )
    def _sc_degree(zeros_hbm, ones_hbm, ei_hbm, out_hbm, src_o, dst_o,
                   srcv, dstv, onesv, bounce, acc, lsem, ssem):
        c = lax.axis_index("c")
        s = lax.axis_index("s")
        wid = _wid()
        base = s * ROWS_PER_S
        ebase = wid * PER_W
        for j in range(FULL_ROWS):
            pltpu.async_copy(
                ei_hbm.at[0, pl.ds(ebase + j * BB, BB)], srcv.at[j], lsem)
            pltpu.async_copy(
                ei_hbm.at[1, pl.ds(ebase + j * BB, BB)], dstv.at[j], ssem)
        pltpu.async_copy(
            ei_hbm.at[0, pl.ds(ebase + FULL_ROWS * BB, TAIL)],
            srcv.at[FULL_ROWS, pl.ds(0, TAIL)], lsem)
        pltpu.async_copy(
            ei_hbm.at[1, pl.ds(ebase + FULL_ROWS * BB, TAIL)],
            dstv.at[FULL_ROWS, pl.ds(0, TAIL)], ssem)
        pltpu.sync_copy(zeros_hbm, bounce)
        pltpu.sync_copy(bounce, acc.at[pl.ds(base, ROWS_PER_S)])
        pltpu.sync_copy(ones_hbm, onesv)
        for j in range(FULL_ROWS):
            pltpu.make_async_copy(
                ei_hbm.at[0, pl.ds(ebase + j * BB, BB)], srcv.at[j],
                lsem).wait()
            pltpu.make_async_copy(
                ei_hbm.at[1, pl.ds(ebase + j * BB, BB)], dstv.at[j],
                ssem).wait()
        pltpu.make_async_copy(
            ei_hbm.at[0, pl.ds(ebase + FULL_ROWS * BB, TAIL)],
            srcv.at[FULL_ROWS, pl.ds(0, TAIL)], lsem).wait()
        pltpu.make_async_copy(
            ei_hbm.at[1, pl.ds(ebase + FULL_ROWS * BB, TAIL)],
            dstv.at[FULL_ROWS, pl.ds(0, TAIL)], ssem).wait()
        iota = lax.iota(jnp.int32, LANES)
        for k in range(1, BB // LANES):
            trash = TRASH + iota + (LANES * (k - 1)) % (NP - N_NODES - LANES)
            srcv[FULL_ROWS, pl.ds(k * LANES, LANES)] = trash
            dstv[FULL_ROWS, pl.ds(k * LANES, LANES)] = trash
        head_s = srcv[FULL_ROWS, pl.ds(0, LANES)]
        head_d = dstv[FULL_ROWS, pl.ds(0, LANES)]
        tail_trash = TRASH + iota + 112
        srcv[FULL_ROWS, pl.ds(0, LANES)] = jnp.where(
            iota < TAIL, head_s, tail_trash)
        dstv[FULL_ROWS, pl.ds(0, LANES)] = jnp.where(
            iota < TAIL, head_d, tail_trash)
        pltpu.async_copy(srcv, src_o.at[wid], lsem)
        pltpu.async_copy(dstv, dst_o.at[wid], lsem)
        plsc.subcore_barrier()

        for jo in range(NB // NBUF):
            for b in range(NBUF):
                j = jo * NBUF + b
                pltpu.async_copy(onesv, acc.at[srcv.at[j]], ssem, add=True)
            for b in range(NBUF):
                j = jo * NBUF + b
                pltpu.make_async_copy(onesv, acc.at[srcv.at[j]], ssem).wait()
        pltpu.make_async_copy(srcv, src_o.at[wid], lsem).wait()
        pltpu.make_async_copy(dstv, dst_o.at[wid], lsem).wait()
        plsc.subcore_barrier()
        pltpu.sync_copy(acc.at[pl.ds(base, ROWS_PER_S)], bounce)
        pltpu.sync_copy(bounce, out_hbm.at[c, pl.ds(base, ROWS_PER_S)])

    return _sc_degree


@functools.cache
def _sc_aggregate_fn():
    mesh = plsc.VectorSubcoreMesh(core_axis_name="c", subcore_axis_name="s",
                                  num_cores=NC, num_subcores=NS)

    @functools.partial(
        pl.kernel,
        out_type=jax.ShapeDtypeStruct((NC, NP, LANES), jnp.float32),
        mesh=mesh,
        compiler_params=pltpu.CompilerParams(use_tc_tiling_on_sc=False),
        scratch_types=[
            pltpu.VMEM((NB, BB), jnp.int32),
            pltpu.VMEM((NB, BB), jnp.int32),
            pltpu.VMEM((NBUF, BB, LANES), jnp.float32),
            pltpu.VMEM((ROWS_PER_S, LANES), jnp.float32),
            pltpu.VMEM_SHARED((NP, LANES), jnp.float32),
            pltpu.SemaphoreType.DMA((NBUF,)),
            pltpu.SemaphoreType.DMA((NBUF,)),
        ],
    )
    def _sc_aggregate(g_hbm, src_hbm, dst_hbm, out_hbm, srcv, dstv, bufs,
                      bounce, acc, gsem, ssem):
        c = lax.axis_index("c")
        s = lax.axis_index("s")
        base = s * ROWS_PER_S
        pltpu.async_copy(src_hbm.at[_wid()], srcv, gsem.at[0])
        pltpu.async_copy(dst_hbm.at[_wid()], dstv, gsem.at[1])
        pltpu.async_copy(g_hbm.at[pl.ds(base, ROWS_PER_S)], bounce, gsem.at[2])
        pltpu.make_async_copy(
            g_hbm.at[pl.ds(base, ROWS_PER_S)], bounce, gsem.at[2]).wait()
        pltpu.sync_copy(bounce, acc.at[pl.ds(base, ROWS_PER_S)])
        pltpu.make_async_copy(src_hbm.at[_wid()], srcv, gsem.at[0]).wait()
        pltpu.make_async_copy(dst_hbm.at[_wid()], dstv, gsem.at[1]).wait()
        plsc.subcore_barrier()

        for b in range(NBUF):
            pltpu.async_copy(g_hbm.at[srcv.at[b]], bufs.at[b], gsem.at[b])

        for jo in range(NB // NBUF - 1):
            for b in range(NBUF):
                j = jo * NBUF + b
                pltpu.make_async_copy(
                    g_hbm.at[srcv.at[j]], bufs.at[b], gsem.at[b]).wait()
                pltpu.async_copy(
                    bufs.at[b], acc.at[dstv.at[j]], ssem.at[b], add=True)
            for b in range(NBUF):
                j = jo * NBUF + b
                pltpu.make_async_copy(
                    bufs.at[b], acc.at[dstv.at[j]], ssem.at[b]).wait()
                pltpu.async_copy(
                    g_hbm.at[srcv.at[j + NBUF]], bufs.at[b], gsem.at[b])
        for b in range(NBUF):
            j = NB - NBUF + b
            pltpu.make_async_copy(
                g_hbm.at[srcv.at[j]], bufs.at[b], gsem.at[b]).wait()
            pltpu.async_copy(
                bufs.at[b], acc.at[dstv.at[j]], ssem.at[b], add=True)
        for b in range(NBUF):
            j = NB - NBUF + b
            pltpu.make_async_copy(
                bufs.at[b], acc.at[dstv.at[j]], ssem.at[b]).wait()
        plsc.subcore_barrier()
        pltpu.sync_copy(acc.at[pl.ds(base, ROWS_PER_S)], bounce)
        pltpu.sync_copy(bounce, out_hbm.at[c, pl.ds(base, ROWS_PER_S)])

    return _sc_aggregate


NPK = NP // 8


def _tc_matmul1_body(x_ref, w1_ref, b1_ref, lin_ref):
    lin = lax.dot_general(x_ref[...], w1_ref[...], (((1,), (1,)), ((), ())),
                          preferred_element_type=jnp.float32) + b1_ref[...]
    lin_ref[pl.ds(0, N_NODES), :] = lin
    lin_ref[pl.ds(N_NODES, NP - N_NODES), :] = jnp.zeros(
        (NP - N_NODES, LANES), jnp.float32)


def _tc_matmul1(x, W1, b1):
    return pl.pallas_call(
        _tc_matmul1_body,
        out_shape=jax.ShapeDtypeStruct((NP, LANES), jnp.float32),
    )(x, W1, b1)


def _disp(degp_ref):
    return lax.rsqrt(1.0 + degp_ref[0] + degp_ref[1])


def _tc_scale1_body(lin_ref, degp_ref, g1_ref):
    g1_ref[...] = lin_ref[...] * _disp(degp_ref)


def _tc_scale1(linp, degpp):
    return pl.pallas_call(
        _tc_scale1_body,
        out_shape=jax.ShapeDtypeStruct((NPK, 128), jnp.float32),
    )(linp, degpp)


def _tc_fuse2_body(g1_ref, p_ref, degp_ref, bw2_ref, b2_ref, g2_ref):
    dis = _disp(degp_ref)
    h1 = jnp.maximum(dis * (p_ref[0] + p_ref[1] - g1_ref[...]), 0.0)
    lin = lax.dot_general(h1, bw2_ref[...], (((1,), (0,)), ((), ())),
                          preferred_element_type=jnp.float32) + b2_ref[...]
    g2_ref[...] = lin * dis


def _tc_fuse2(g1p, pp, degpp, BW2, b2t):
    return pl.pallas_call(
        _tc_fuse2_body,
        out_shape=jax.ShapeDtypeStruct((NPK, 128), jnp.float32),
    )(g1p, pp, degpp, BW2, b2t)


def _tc_fuse3_body(g2_ref, q_ref, degp_ref, bones_ref, mask_ref, out_ref):
    mask = mask_ref[...]
    bones = bones_ref[...]
    z = _disp(degp_ref) * (q_ref[0] + q_ref[1] - g2_ref[...])
    gsum = lax.dot_general(z * mask, bones, (((1,), (0,)), ((), ())),
                           preferred_element_type=jnp.float32)
    m = gsum * (1.0 / N_CLASSES)
    e = jnp.exp(jnp.clip(z - m, -80.0, 80.0)) * mask
    se = lax.dot_general(e, bones, (((1,), (0,)), ((), ())),
                         preferred_element_type=jnp.float32)
    out_ref[...] = z - m - jnp.log(se)


def _tc_fuse3(g2p, qp, degpp, BO, maskp):
    return pl.pallas_call(
        _tc_fuse3_body,
        out_shape=jax.ShapeDtypeStruct((NPK, 128), jnp.float32),
    )(g2p, qp, degpp, BO, maskp)


def kernel(x, edge_index, W1, b1, W2, b2):
    ei32 = edge_index.astype(jnp.int32)
    zeros = jnp.zeros((ROWS_PER_S, LANES), jnp.float32)
    ones = jnp.ones((BB, LANES), jnp.float32)
    W2p = jnp.zeros((LANES, HIDDEN), jnp.float32).at[:N_CLASSES].set(W2)
    BW2 = jnp.kron(jnp.eye(8, dtype=jnp.float32), W2p.T)
    b2t = jnp.tile(jnp.zeros((LANES,), jnp.float32).at[:N_CLASSES].set(b2),
                   8).reshape(1, 128)
    BO = jnp.kron(jnp.eye(8, dtype=jnp.float32),
                  jnp.ones((LANES, LANES), jnp.float32))
    maskp = jnp.tile(
        (jnp.arange(LANES) < N_CLASSES).astype(jnp.float32), 8).reshape(1, 128)
    b1r = b1.reshape(1, HIDDEN)

    def pack(a):
        return a.reshape(a.shape[:-2] + (NPK, 128))

    def unpack(a):
        return a.reshape(a.shape[:-2] + (NP, LANES))

    sc_degree = _sc_degree_fn()
    sc_aggregate = _sc_aggregate_fn()
    degp, src_p, dst_p = sc_degree(zeros, ones, ei32)
    degpp = pack(degp)
    linp = pack(_tc_matmul1(x, W1, b1r))
    g1p = _tc_scale1(linp, degpp)
    pp = pack(sc_aggregate(unpack(g1p), src_p, dst_p))
    g2p = _tc_fuse2(g1p, pp, degpp, BW2, b2t)
    qp = pack(sc_aggregate(unpack(g2p), src_p, dst_p))
    outp = _tc_fuse3(g2p, qp, degpp, BO, maskp)
    return outp[:N_NODES // 8].reshape(N_NODES, LANES)[:, :N_CLASSES]
